# baseline (speedup 1.0000x reference)
import jax
import jax.numpy as jnp
from jax.experimental import pallas as pl
from jax.experimental.pallas import tpu as pltpu

B, T, D, H, W = 4, 16, 192, 56, 56


def _body(t_map_ref, x_ref, pe_ref, o_ref):
    pev = pe_ref[0, 0, :]
    o_ref[...] = x_ref[...] + pev[None, None, :, None, None]


def kernel(x, t_idx, pe):
    pe_r = pe.reshape(1000, 1, D)
    grid = (T, B)
    out = pl.pallas_call(
        _body,
        grid_spec=pltpu.PrefetchScalarGridSpec(
            num_scalar_prefetch=1,
            grid=grid,
            in_specs=[
                pl.BlockSpec((1, 1, D, H, W), lambda t, b, t_map: (b, t, 0, 0, 0)),
                pl.BlockSpec((1, 1, D), lambda t, b, t_map: (t_map[t], 0, 0)),
            ],
            out_specs=pl.BlockSpec((1, 1, D, H, W), lambda t, b, t_map: (b, t, 0, 0, 0)),
        ),
        out_shape=jax.ShapeDtypeStruct(x.shape, x.dtype),
        compiler_params=pltpu.CompilerParams(
            dimension_semantics=("arbitrary", "arbitrary"),
        ),
    )(t_idx.astype(jnp.int32), x, pe_r)
    return out


# speedup vs baseline: 1.7384x; 1.7384x over previous
import jax
import jax.numpy as jnp
from jax.experimental import pallas as pl
from jax.experimental.pallas import tpu as pltpu

B, T, D, H, W = 4, 16, 192, 56, 56


def _body(t_map_ref, x_ref, pe_ref, o_ref):
    pev = pe_ref[0, 0, :]
    o_ref[...] = x_ref[...] + pev[None, None, :, None]


def kernel(x, t_idx, pe):
    xr = x.reshape(B, T, D, H * W)
    pe_r = pe.reshape(1000, 1, D)
    grid = (T, B)
    out = pl.pallas_call(
        _body,
        grid_spec=pltpu.PrefetchScalarGridSpec(
            num_scalar_prefetch=1,
            grid=grid,
            in_specs=[
                pl.BlockSpec((1, 1, D, H * W), lambda t, b, t_map: (b, t, 0, 0)),
                pl.BlockSpec((1, 1, D), lambda t, b, t_map: (t_map[t], 0, 0)),
            ],
            out_specs=pl.BlockSpec((1, 1, D, H * W), lambda t, b, t_map: (b, t, 0, 0)),
        ),
        out_shape=jax.ShapeDtypeStruct(xr.shape, xr.dtype),
        compiler_params=pltpu.CompilerParams(
            dimension_semantics=("arbitrary", "arbitrary"),
        ),
    )(t_idx.astype(jnp.int32), xr, pe_r)
    return out.reshape(B, T, D, H, W)
